# baseline (speedup 1.0000x reference)
import functools

import jax
import jax.numpy as jnp
from jax import lax
from jax.experimental import pallas as pl
from jax.experimental.pallas import tpu as pltpu
from jax.experimental.pallas import tpu_sc as plsc

_LANES = 16
_TILES = 16
_DECAY = 0.99


def _make_sc_argmin(m):
    chunk = m // _TILES
    nvec = chunk // _LANES
    mesh = plsc.VectorSubcoreMesh(
        core_axis_name="c", subcore_axis_name="s", num_cores=1)

    @functools.partial(
        pl.kernel,
        out_type=[
            jax.ShapeDtypeStruct((m,), jnp.float32),
            jax.ShapeDtypeStruct((_LANES,), jnp.int32),
        ],
        mesh=mesh,
        scratch_types=[
            pltpu.VMEM((chunk,), jnp.float32),
            pltpu.VMEM((chunk,), jnp.float32),
            pltpu.VMEM((_LANES,), jnp.float32),
            pltpu.VMEM((_LANES,), jnp.int32),
            pltpu.VMEM_SHARED((_TILES * _LANES,), jnp.float32),
            pltpu.VMEM_SHARED((_TILES * _LANES,), jnp.int32),
            pltpu.VMEM((_TILES * _LANES,), jnp.float32),
            pltpu.VMEM((_TILES * _LANES,), jnp.int32),
            pltpu.VMEM((_LANES,), jnp.int32),
        ],
    )
    def sc_argmin(usage_hbm, new_usage_hbm, idx_hbm,
                  u_v, o_v, st_min, st_idx, sh_min, sh_idx,
                  gb_min, gb_idx, idx_v):
        wid = lax.axis_index("s")
        base = wid * chunk
        pltpu.sync_copy(usage_hbm.at[pl.ds(base, chunk)], u_v)
        lanes = lax.iota(jnp.int32, _LANES)

        def scan_body(i, carry):
            vmin, vidx = carry
            off = pl.multiple_of(i * _LANES, _LANES)
            v = u_v[pl.ds(off, _LANES)]
            ids = base + i * _LANES + lanes
            take = v < vmin
            return (jnp.where(take, v, vmin), jnp.where(take, ids, vidx))

        vmin, vidx = lax.fori_loop(
            0, nvec, scan_body,
            (jnp.full((_LANES,), jnp.inf, jnp.float32),
             jnp.zeros((_LANES,), jnp.int32)))

        st_min[...] = vmin
        st_idx[...] = vidx
        pltpu.sync_copy(st_min, sh_min.at[pl.ds(wid * _LANES, _LANES)])
        pltpu.sync_copy(st_idx, sh_idx.at[pl.ds(wid * _LANES, _LANES)])
        plsc.subcore_barrier()
        pltpu.sync_copy(sh_min, gb_min)
        pltpu.sync_copy(sh_idx, gb_idx)

        g_min = jnp.full((_LANES,), jnp.inf, jnp.float32)
        g_idx = jnp.zeros((_LANES,), jnp.int32)
        for j in range(_TILES):
            vj = gb_min[pl.ds(j * _LANES, _LANES)]
            ij = gb_idx[pl.ds(j * _LANES, _LANES)]
            take = vj < g_min
            g_min = jnp.where(take, vj, g_min)
            g_idx = jnp.where(take, ij, g_idx)

        bv = jnp.float32(jnp.inf)
        bi = jnp.int32(2**31 - 1)
        for j in range(_LANES):
            v = g_min[j]
            ix = g_idx[j]
            upd = (v < bv) | ((v == bv) & (ix < bi))
            bv = jnp.where(upd, v, bv)
            bi = jnp.where(upd, ix, bi)
        g_idx = jnp.full((_LANES,), bi, jnp.int32)

        def out_body(i, _):
            off = pl.multiple_of(i * _LANES, _LANES)
            u = u_v[pl.ds(off, _LANES)]
            ids = base + i * _LANES + lanes
            hit = ids == g_idx
            o_v[pl.ds(off, _LANES)] = jnp.where(
                hit, jnp.float32(_DECAY), u * jnp.float32(_DECAY))
            return 0

        lax.fori_loop(0, nvec, out_body, 0)
        pltpu.sync_copy(o_v, new_usage_hbm.at[pl.ds(base, chunk)])

        @pl.when(wid == 0)
        def _():
            idx_v[...] = g_idx
            pltpu.sync_copy(idx_v, idx_hbm)

    return sc_argmin


def _make_tc(bq, ks, vs, m, mb):
    grid = (m // mb,)

    def body(idx_ref, q_ref, upk_ref, upv_ref, k_ref, v_ref,
             read_ref, nk_ref, nv_ref):
        i = pl.program_id(0)
        q = q_ref[...]
        k = k_ref[...]
        v = v_ref[...]
        sims = lax.dot_general(q, k, (((1,), (1,)), ((), ())),
                               preferred_element_type=jnp.float32)
        w = jax.nn.sigmoid(sims)
        contrib = jnp.dot(w, v, preferred_element_type=jnp.float32)

        @pl.when(i == 0)
        def _():
            read_ref[...] = contrib

        @pl.when(i > 0)
        def _():
            read_ref[...] += contrib

        idx = idx_ref[0]
        rows = lax.broadcasted_iota(jnp.int32, (mb, 1), 0) + i * mb
        hit = rows == idx
        nk_ref[...] = jnp.where(hit, upk_ref[...], k)
        nv_ref[...] = jnp.where(hit, upv_ref[...], v)

    return pl.pallas_call(
        body,
        grid=grid,
        in_specs=[
            pl.BlockSpec(memory_space=pltpu.SMEM),
            pl.BlockSpec((bq, ks), lambda i: (0, 0)),
            pl.BlockSpec((1, ks), lambda i: (0, 0)),
            pl.BlockSpec((1, vs), lambda i: (0, 0)),
            pl.BlockSpec((mb, ks), lambda i: (i, 0)),
            pl.BlockSpec((mb, vs), lambda i: (i, 0)),
        ],
        out_specs=[
            pl.BlockSpec((bq, vs), lambda i: (0, 0)),
            pl.BlockSpec((mb, ks), lambda i: (i, 0)),
            pl.BlockSpec((mb, vs), lambda i: (i, 0)),
        ],
        out_shape=[
            jax.ShapeDtypeStruct((bq, vs), jnp.float32),
            jax.ShapeDtypeStruct((m, ks), jnp.float32),
            jax.ShapeDtypeStruct((m, vs), jnp.float32),
        ],
        compiler_params=pltpu.CompilerParams(
            dimension_semantics=("arbitrary",)),
    )


def kernel(query_key, upd_key, upd_value, mem_keys, mem_values, usage):
    m, ks = mem_keys.shape
    vs = mem_values.shape[1]
    bq = query_key.shape[0]

    new_usage, idx16 = _make_sc_argmin(m)(usage)
    read, new_keys, new_values = _make_tc(bq, ks, vs, m, 2048)(
        idx16, query_key, upd_key, upd_value, mem_keys, mem_values)
    return read, new_keys, new_values, new_usage


# speedup vs baseline: 1.3374x; 1.3374x over previous
import functools

import jax
import jax.numpy as jnp
from jax import lax
from jax.experimental import pallas as pl
from jax.experimental.pallas import tpu as pltpu
from jax.experimental.pallas import tpu_sc as plsc

_LANES = 16
_TILES = 16
_DECAY = 0.99


def _make_sc_argmin(m):
    chunk = m // _TILES
    nvec = chunk // _LANES
    mesh = plsc.VectorSubcoreMesh(
        core_axis_name="c", subcore_axis_name="s", num_cores=1)

    @functools.partial(
        pl.kernel,
        out_type=[
            jax.ShapeDtypeStruct((m,), jnp.float32),
            jax.ShapeDtypeStruct((_LANES,), jnp.int32),
        ],
        mesh=mesh,
        scratch_types=[
            pltpu.VMEM((chunk,), jnp.float32),
            pltpu.VMEM((chunk,), jnp.float32),
            pltpu.VMEM((_LANES,), jnp.float32),
            pltpu.VMEM((_LANES,), jnp.int32),
            pltpu.VMEM_SHARED((_TILES * _LANES,), jnp.float32),
            pltpu.VMEM_SHARED((_TILES * _LANES,), jnp.int32),
            pltpu.VMEM((_TILES * _LANES,), jnp.float32),
            pltpu.VMEM((_TILES * _LANES,), jnp.int32),
            pltpu.VMEM((_LANES,), jnp.int32),
        ],
    )
    def sc_argmin(usage_hbm, new_usage_hbm, idx_hbm,
                  u_v, o_v, st_min, st_idx, sh_min, sh_idx,
                  gb_min, gb_idx, idx_v):
        wid = lax.axis_index("s")
        base = wid * chunk
        pltpu.sync_copy(usage_hbm.at[pl.ds(base, chunk)], u_v)
        lanes = lax.iota(jnp.int32, _LANES)

        def scan_body(i, carry):
            vmin, vidx = carry
            off = pl.multiple_of(i * _LANES, _LANES)
            v = u_v[pl.ds(off, _LANES)]
            ids = base + i * _LANES + lanes
            take = v < vmin
            return (jnp.where(take, v, vmin), jnp.where(take, ids, vidx))

        vmin, vidx = lax.fori_loop(
            0, nvec, scan_body,
            (jnp.full((_LANES,), jnp.inf, jnp.float32),
             jnp.zeros((_LANES,), jnp.int32)))

        st_min[...] = vmin
        st_idx[...] = vidx
        pltpu.sync_copy(st_min, sh_min.at[pl.ds(wid * _LANES, _LANES)])
        pltpu.sync_copy(st_idx, sh_idx.at[pl.ds(wid * _LANES, _LANES)])
        plsc.subcore_barrier()
        pltpu.sync_copy(sh_min, gb_min)
        pltpu.sync_copy(sh_idx, gb_idx)

        g_min = jnp.full((_LANES,), jnp.inf, jnp.float32)
        g_idx = jnp.zeros((_LANES,), jnp.int32)
        for j in range(_TILES):
            vj = gb_min[pl.ds(j * _LANES, _LANES)]
            ij = gb_idx[pl.ds(j * _LANES, _LANES)]
            take = vj < g_min
            g_min = jnp.where(take, vj, g_min)
            g_idx = jnp.where(take, ij, g_idx)

        bv = jnp.float32(jnp.inf)
        bi = jnp.int32(2**31 - 1)
        for j in range(_LANES):
            v = g_min[j]
            ix = g_idx[j]
            upd = (v < bv) | ((v == bv) & (ix < bi))
            bv = jnp.where(upd, v, bv)
            bi = jnp.where(upd, ix, bi)
        g_idx = jnp.full((_LANES,), bi, jnp.int32)

        def out_body(i, _):
            off = pl.multiple_of(i * _LANES, _LANES)
            u = u_v[pl.ds(off, _LANES)]
            ids = base + i * _LANES + lanes
            hit = ids == g_idx
            o_v[pl.ds(off, _LANES)] = jnp.where(
                hit, jnp.float32(_DECAY), u * jnp.float32(_DECAY))
            return 0

        lax.fori_loop(0, nvec, out_body, 0)
        pltpu.sync_copy(o_v, new_usage_hbm.at[pl.ds(base, chunk)])

        @pl.when(wid == 0)
        def _():
            idx_v[...] = g_idx
            pltpu.sync_copy(idx_v, idx_hbm)

    return sc_argmin


def _make_tc(bq, ks, vs, m, mb):
    grid = (m // mb,)

    def body(idx_ref, qt_ref, upkt_ref, upv_ref, kt_ref, v_ref,
             read_ref, nkt_ref, nv_ref):
        i = pl.program_id(0)
        qt = qt_ref[...]
        kt = kt_ref[...]
        v = v_ref[...]
        sims = lax.dot_general(qt, kt, (((0,), (0,)), ((), ())),
                               preferred_element_type=jnp.float32)
        w = 0.5 * jnp.tanh(sims * 0.5) + 0.5
        contrib = jnp.dot(w, v, preferred_element_type=jnp.float32)

        @pl.when(i == 0)
        def _():
            read_ref[...] = contrib

        @pl.when(i > 0)
        def _():
            read_ref[...] += contrib

        idx = idx_ref[0]
        cols = lax.broadcasted_iota(jnp.int32, (1, mb), 1) + i * mb
        hit_c = cols == idx
        nkt_ref[...] = jnp.where(hit_c, upkt_ref[...], kt)
        rows = lax.broadcasted_iota(jnp.int32, (mb, 1), 0) + i * mb
        hit_r = rows == idx
        nv_ref[...] = jnp.where(hit_r, upv_ref[...], v)

    return pl.pallas_call(
        body,
        grid=grid,
        in_specs=[
            pl.BlockSpec(memory_space=pltpu.SMEM),
            pl.BlockSpec((ks, bq), lambda i: (0, 0)),
            pl.BlockSpec((ks, 1), lambda i: (0, 0)),
            pl.BlockSpec((1, vs), lambda i: (0, 0)),
            pl.BlockSpec((ks, mb), lambda i: (0, i)),
            pl.BlockSpec((mb, vs), lambda i: (i, 0)),
        ],
        out_specs=[
            pl.BlockSpec((bq, vs), lambda i: (0, 0)),
            pl.BlockSpec((ks, mb), lambda i: (0, i)),
            pl.BlockSpec((mb, vs), lambda i: (i, 0)),
        ],
        out_shape=[
            jax.ShapeDtypeStruct((bq, vs), jnp.float32),
            jax.ShapeDtypeStruct((ks, m), jnp.float32),
            jax.ShapeDtypeStruct((m, vs), jnp.float32),
        ],
        compiler_params=pltpu.CompilerParams(
            dimension_semantics=("arbitrary",)),
    )


def kernel(query_key, upd_key, upd_value, mem_keys, mem_values, usage):
    m, ks = mem_keys.shape
    vs = mem_values.shape[1]
    bq = query_key.shape[0]

    new_usage, idx16 = _make_sc_argmin(m)(usage)
    read, new_keys_t, new_values = _make_tc(bq, ks, vs, m, 2048)(
        idx16, query_key.T, upd_key.T, upd_value, mem_keys.T, mem_values)
    return read, new_keys_t.T, new_values, new_usage
